# baseline (speedup 1.0000x reference)
import functools

import jax
import jax.numpy as jnp
from jax import lax
from jax.experimental import pallas as pl
from jax.experimental.pallas import tpu as pltpu
from jax.experimental.pallas import tpu_sc as plsc

N = 10000
E = 160000
IN = 256
H = 512
OUT = 128
G = 64
EPS = 1e-5
_BNS = (1.0 + EPS) ** -0.5

_CW = 128
_CB = 128
_NSUB = 16
_EPAD = 1280 * _CB
_KPT = _EPAD // _CB // _NSUB
_HN = N // 2
_TRASH = _HN
_AROWS = 5120
_RB = _AROWS // _NSUB
_SB = 64
_WLAST = _HN - (_NSUB - 1) * _RB
_R = 1000
_NB = N // _R


def _zero_acc(sid, acc, zbuf):
    for q in range(_RB // _SB):
        pltpu.sync_copy(zbuf, acc.at[pl.ds(sid * _RB + q * _SB, _SB)])


def _writeback(sid, acc, wbuf, out, base):
    def blk(r0, nr, woff):
        pltpu.sync_copy(acc.at[pl.ds(r0, nr)], wbuf.at[pl.ds(woff, nr)])
        pltpu.sync_copy(wbuf.at[pl.ds(woff, nr)], out.at[pl.ds(base + r0, nr)])

    @pl.when(sid < _NSUB - 1)
    def _():
        for q in range(_RB // _SB):
            blk(sid * _RB + q * _SB, _SB, 0)

    @pl.when(sid == _NSUB - 1)
    def _():
        r0 = (_NSUB - 1) * _RB
        for q in range(_WLAST // _SB):
            blk(r0 + q * _SB, _SB, 0)
        tail = _WLAST - (_WLAST // _SB) * _SB
        if tail:
            blk(r0 + (_WLAST // _SB) * _SB, tail, 0)


def _edge_pass(sid, sh, dh, table, acc, srcv, dstv, rows, sem,
               onesv=None, cacc=None):
    def body(k, carry):
        off = (sid + k * _NSUB) * _CB
        pltpu.sync_copy(sh.at[pl.ds(off, _CB)], srcv)
        pltpu.sync_copy(dh.at[pl.ds(off, _CB)], dstv)
        pltpu.async_copy(table.at[srcv], rows, sem).wait()
        pltpu.sync_copy(rows, acc.at[dstv], add=True)
        if cacc is not None:
            pltpu.sync_copy(onesv, cacc.at[dstv], add=True)
        return carry

    lax.fori_loop(0, _KPT, body, 0)


def _sc_segsum(tables, srcP, dloP, dhiP, zeros_in, ones128, with_counts):
    npass = len(tables)
    ntot = npass + (1 if with_counts else 0)
    mesh = plsc.VectorSubcoreMesh(core_axis_name="c", subcore_axis_name="s")
    out_type = [jax.ShapeDtypeStruct((N, _CW), jnp.float32)] * ntot
    scratch = [
        pltpu.VMEM((_CB,), jnp.int32),
        pltpu.VMEM((_CB,), jnp.int32),
        pltpu.VMEM((_CB, _CW), jnp.float32),
        pltpu.VMEM((_SB, _CW), jnp.float32),
        pltpu.VMEM((_SB, _CW), jnp.float32),
        pltpu.VMEM_SHARED((_AROWS, _CW), jnp.float32),
        pltpu.SemaphoreType.DMA,
    ]

    @functools.partial(pl.kernel, out_type=out_type, mesh=mesh,
                       scratch_types=scratch)
    def k(*refs):
        tabs = refs[:npass]
        sh, dlo, dhi, zh, onesh = refs[npass:npass + 5]
        s_outs = refs[npass + 5:npass + 5 + ntot]
        (srcv, dstv, rows, zbuf, wbuf, acc, sem) = refs[npass + 5 + ntot:]

        cid = lax.axis_index("c")
        sid = lax.axis_index("s")
        base = cid * _HN
        pltpu.sync_copy(zh, zbuf)

        for p in range(npass):
            _zero_acc(sid, acc, zbuf)
            plsc.subcore_barrier()

            @pl.when(cid == 0)
            def _(tab=tabs[p]):
                _edge_pass(sid, sh, dlo, tab, acc, srcv, dstv, rows, sem)

            @pl.when(cid == 1)
            def _(tab=tabs[p]):
                _edge_pass(sid, sh, dhi, tab, acc, srcv, dstv, rows, sem)
            plsc.subcore_barrier()
            _writeback(sid, acc, wbuf, s_outs[p], base)
            plsc.subcore_barrier()

        if with_counts:
            _zero_acc(sid, acc, zbuf)
            pltpu.sync_copy(onesh, rows)
            plsc.subcore_barrier()

            def cbody(kk, carry, dh):
                off = (sid + kk * _NSUB) * _CB
                pltpu.sync_copy(dh.at[pl.ds(off, _CB)], dstv)
                pltpu.sync_copy(rows, acc.at[dstv], add=True)
                return carry

            @pl.when(cid == 0)
            def _():
                lax.fori_loop(0, _KPT, functools.partial(cbody, dh=dlo), 0)

            @pl.when(cid == 1)
            def _():
                lax.fori_loop(0, _KPT, functools.partial(cbody, dh=dhi), 0)
            plsc.subcore_barrier()
            _writeback(sid, acc, wbuf, s_outs[npass], base)

    args = list(tables) + [srcP, dloP, dhiP, zeros_in, ones128]
    res = k(*args)
    return tuple(res) if isinstance(res, (list, tuple)) else (res,)


_PROBE_STAGE = 3


def _sc_probe(table, srcP, dloP, zeros_in):
    mesh = plsc.VectorSubcoreMesh(core_axis_name="c", subcore_axis_name="s")
    scratch = [
        pltpu.VMEM((_CB,), jnp.int32),
        pltpu.VMEM((_CB,), jnp.int32),
        pltpu.VMEM((_CB, _CW), jnp.float32),
        pltpu.VMEM((_SB, _CW), jnp.float32),
        pltpu.VMEM((_SB, _CW), jnp.float32),
        pltpu.VMEM_SHARED((_AROWS, _CW), jnp.float32),
        pltpu.SemaphoreType.DMA,
    ]

    @functools.partial(
        pl.kernel,
        out_type=jax.ShapeDtypeStruct((N, _CW), jnp.float32),
        mesh=mesh, scratch_types=scratch)
    def k(tab, sh, dlo, zh, s_out, srcv, dstv, rows, zbuf, wbuf, acc, sem):
        cid = lax.axis_index("c")
        sid = lax.axis_index("s")
        base = cid * _HN
        pltpu.sync_copy(zh, zbuf)
        _zero_acc(sid, acc, zbuf)
        plsc.subcore_barrier()

        if _PROBE_STAGE >= 2:
            def body(kk, carry):
                off = (sid + kk * _NSUB) * _CB
                pltpu.sync_copy(sh.at[pl.ds(off, _CB)], srcv)
                pltpu.sync_copy(dlo.at[pl.ds(off, _CB)], dstv)
                pltpu.async_copy(tab.at[srcv], rows, sem).wait()
                if _PROBE_STAGE >= 3:
                    pltpu.sync_copy(rows, acc.at[dstv], add=True)
                return carry

            lax.fori_loop(0, _KPT, body, 0)
        plsc.subcore_barrier()
        _writeback(sid, acc, wbuf, s_out, base)

    return k(table, srcP, dloP, zeros_in)


def _tc_localize(dstP):
    def body(d_ref, lo_ref, hi_ref):
        d = d_ref[...]
        lo_ref[...] = jnp.where(d < _HN, d, _TRASH)
        hi_ref[...] = jnp.where((d >= _HN) & (d < N), d - _HN, _TRASH)

    return pl.pallas_call(
        body,
        out_shape=[jax.ShapeDtypeStruct((_EPAD // _CB, _CB), jnp.int32)] * 2,
    )(dstP)


def _tc_layer0(s0, cntb, x, Wl0T, Wr0T, bl0, g0, b0):
    def body(s0_ref, cnt_ref, x_ref, wl_ref, wr_ref, bl_ref, g_ref, b_ref,
             ofull, *ochunks):
        inv = 1.0 / jnp.maximum(cnt_ref[...], 1.0)
        zl = jnp.dot(s0_ref[...], wl_ref[...],
                     preferred_element_type=jnp.float32)
        zr = jnp.dot(x_ref[...], wr_ref[...],
                     preferred_element_type=jnp.float32)
        z = jnp.concatenate(
            [zl[:, c * 128:(c + 1) * 128] * inv for c in range(H // 128)],
            axis=1)
        z = ((z + zr + bl_ref[...]) * (g_ref[...] * _BNS)) + b_ref[...]
        h = jnp.maximum(z, 0.0)
        ofull[...] = h
        for j, o in enumerate(ochunks):
            o[...] = h[:, j * _CW:(j + 1) * _CW]

    return pl.pallas_call(
        body,
        grid=(_NB,),
        in_specs=[
            pl.BlockSpec((_R, IN), lambda i: (i, 0)),
            pl.BlockSpec((_R, 128), lambda i: (i, 0)),
            pl.BlockSpec((_R, IN), lambda i: (i, 0)),
            pl.BlockSpec((IN, H), lambda i: (0, 0)),
            pl.BlockSpec((IN, H), lambda i: (0, 0)),
            pl.BlockSpec((1, H), lambda i: (0, 0)),
            pl.BlockSpec((1, H), lambda i: (0, 0)),
            pl.BlockSpec((1, H), lambda i: (0, 0)),
        ],
        out_specs=[pl.BlockSpec((_R, H), lambda i: (i, 0))] +
                  [pl.BlockSpec((_R, _CW), lambda i: (i, 0))] * (H // _CW),
        out_shape=[jax.ShapeDtypeStruct((N, H), jnp.float32)] +
                  [jax.ShapeDtypeStruct((N, _CW), jnp.float32)] * (H // _CW),
    )(s0, cntb, x, Wl0T, Wr0T, bl0, g0, b0)


def _tc_layer1(s1, hfull, cntb, batch3, Wl1T, Wr1T, bl1, g1, b1, WlinT, blin):
    def body(s1_ref, h_ref, cnt_ref, bt_ref, wl_ref, wr_ref,
             bl_ref, g_ref, b_ref, wo_ref, bo_ref, out_ref, pacc, pcacc):
        i = pl.program_id(0)
        inv = 1.0 / jnp.maximum(cnt_ref[...], 1.0)
        zl = jnp.dot(s1_ref[...], wl_ref[...],
                     preferred_element_type=jnp.float32)
        zr = jnp.dot(h_ref[...], wr_ref[...],
                     preferred_element_type=jnp.float32)
        z = jnp.concatenate(
            [zl[:, c * 128:(c + 1) * 128] * inv for c in range(H // 128)],
            axis=1)
        z = ((z + zr + bl_ref[...]) * (g_ref[...] * _BNS)) + b_ref[...]
        h = jnp.maximum(z, 0.0)

        bt = bt_ref[0]
        rows_iota = lax.broadcasted_iota(jnp.int32, (G, _R), 0)
        oh = (rows_iota == bt).astype(jnp.float32)
        ps = jnp.dot(oh, h, preferred_element_type=jnp.float32)
        pc = jnp.sum(oh, axis=1, keepdims=True)

        @pl.when(i == 0)
        def _():
            pacc[...] = jnp.zeros_like(pacc)
            pcacc[...] = jnp.zeros_like(pcacc)

        pacc[...] += ps
        pcacc[...] += jnp.broadcast_to(pc, (G, 128))

        @pl.when(i == _NB - 1)
        def _():
            invp = 1.0 / jnp.maximum(pcacc[...], 1.0)
            o = bo_ref[...]
            for c in range(H // 128):
                o += jnp.dot(pacc[:, c * 128:(c + 1) * 128] * invp,
                             wo_ref[c * 128:(c + 1) * 128, :],
                             preferred_element_type=jnp.float32)
            out_ref[...] = o

    return pl.pallas_call(
        body,
        grid=(_NB,),
        in_specs=[
            pl.BlockSpec((_R, H), lambda i: (i, 0)),
            pl.BlockSpec((_R, H), lambda i: (i, 0)),
            pl.BlockSpec((_R, 128), lambda i: (i, 0)),
            pl.BlockSpec((1, 1, _R), lambda i: (i, 0, 0)),
            pl.BlockSpec((H, H), lambda i: (0, 0)),
            pl.BlockSpec((H, H), lambda i: (0, 0)),
            pl.BlockSpec((1, H), lambda i: (0, 0)),
            pl.BlockSpec((1, H), lambda i: (0, 0)),
            pl.BlockSpec((1, H), lambda i: (0, 0)),
            pl.BlockSpec((H, OUT), lambda i: (0, 0)),
            pl.BlockSpec((1, OUT), lambda i: (0, 0)),
        ],
        out_specs=pl.BlockSpec((G, OUT), lambda i: (0, 0)),
        out_shape=jax.ShapeDtypeStruct((G, OUT), jnp.float32),
        scratch_shapes=[pltpu.VMEM((G, H), jnp.float32),
                        pltpu.VMEM((G, 128), jnp.float32)],
    )(s1, hfull, cntb, batch3, Wl1T, Wr1T, bl1, g1, b1, WlinT, blin)


def kernel(x, edge_index, batch, Wl0, bl0, Wr0, Wl1, bl1, Wr1,
           g0, b0, g1, b1, Wlin, blin):
    src = edge_index[0]
    dst = edge_index[1]
    srcP = jnp.concatenate([src, jnp.zeros((_EPAD - E,), jnp.int32)])
    dstP = jnp.concatenate([dst, jnp.full((_EPAD - E,), N, jnp.int32)])
    dlo, dhi = _tc_localize(dstP.reshape(_EPAD // _CB, _CB))
    dlo, dhi = dlo.reshape(_EPAD), dhi.reshape(_EPAD)
    xc = [x[:, j * _CW:(j + 1) * _CW] for j in range(IN // _CW)]
    zeros_in = jnp.zeros((_SB, _CW), jnp.float32)
    ones128 = jnp.ones((_CB, _CW), jnp.float32)

    res0 = _sc_segsum(xc, srcP, dlo, dhi, zeros_in, ones128, True)
    s0 = jnp.concatenate(res0[:IN // _CW], axis=1)
    cntb = res0[IN // _CW]
    h = _tc_layer0(s0, cntb, x, Wl0.T, Wr0.T,
                   bl0[None, :], g0[None, :], b0[None, :])
    hfull, hc = h[0], h[1:]
    res1 = _sc_segsum(list(hc), srcP, dlo, dhi, zeros_in, ones128, False)
    s1 = jnp.concatenate(res1, axis=1)
    out = _tc_layer1(s1, hfull, cntb, batch.reshape(_NB, 1, _R),
                     Wl1.T, Wr1.T, bl1[None, :], g1[None, :], b1[None, :],
                     Wlin.T, blin[None, :])
    return out


# speedup vs baseline: 1.1782x; 1.1782x over previous
import functools

import jax
import jax.numpy as jnp
from jax import lax
from jax.experimental import pallas as pl
from jax.experimental.pallas import tpu as pltpu
from jax.experimental.pallas import tpu_sc as plsc

N = 10000
E = 160000
IN = 256
H = 512
OUT = 128
G = 64
EPS = 1e-5
_BNS = (1.0 + EPS) ** -0.5

_CW = 128
_CB = 128
_NSUB = 16
_EPAD = 1280 * _CB
_KPT = _EPAD // _CB // _NSUB
_HN = N // 2
_TRASH = _HN
_AROWS = 5120
_RB = _AROWS // _NSUB
_SB = 64
_WLAST = _HN - (_NSUB - 1) * _RB
_R = 1000
_NB = N // _R


def _zero_acc(sid, acc, zbuf):
    for q in range(_RB // _SB):
        pltpu.sync_copy(zbuf, acc.at[pl.ds(sid * _RB + q * _SB, _SB)])


def _writeback(sid, acc, wbuf, out, base):
    def blk(r0, nr, woff):
        pltpu.sync_copy(acc.at[pl.ds(r0, nr)], wbuf.at[pl.ds(woff, nr)])
        pltpu.sync_copy(wbuf.at[pl.ds(woff, nr)], out.at[pl.ds(base + r0, nr)])

    @pl.when(sid < _NSUB - 1)
    def _():
        for q in range(_RB // _SB):
            blk(sid * _RB + q * _SB, _SB, 0)

    @pl.when(sid == _NSUB - 1)
    def _():
        r0 = (_NSUB - 1) * _RB
        for q in range(_WLAST // _SB):
            blk(r0 + q * _SB, _SB, 0)
        tail = _WLAST - (_WLAST // _SB) * _SB
        if tail:
            blk(r0 + (_WLAST // _SB) * _SB, tail, 0)


def _edge_pass(sid, sh, dh, table, acc, srcv, dstv, rows, sems):
    def load_and_fire(k, b):
        off = (sid + k * _NSUB) * _CB
        pltpu.sync_copy(sh.at[pl.ds(off, _CB)], srcv.at[b])
        pltpu.sync_copy(dh.at[pl.ds(off, _CB)], dstv.at[b])
        pltpu.async_copy(table.at[srcv.at[b]], rows.at[b], sems[b])

    for b in range(2):
        load_and_fire(b, b)

    def body(k0, carry):
        for b in range(2):
            pltpu.make_async_copy(table.at[srcv.at[b]], rows.at[b],
                                  sems[b]).wait()
            pltpu.sync_copy(rows.at[b], acc.at[dstv.at[b]], add=True)

            @pl.when(k0 + b + 2 < _KPT)
            def _(b=b, k=k0 + b):
                load_and_fire(k + 2, b)
        return carry

    lax.fori_loop(0, _KPT // 2, lambda q, c: body(q * 2, c), 0)


def _sc_segsum(tables, srcP, dloP, dhiP, zeros_in, ones128, with_counts):
    npass = len(tables)
    ntot = npass + (1 if with_counts else 0)
    mesh = plsc.VectorSubcoreMesh(core_axis_name="c", subcore_axis_name="s")
    out_type = [jax.ShapeDtypeStruct((N, _CW), jnp.float32)] * ntot
    scratch = [
        pltpu.VMEM((2, _CB), jnp.int32),
        pltpu.VMEM((2, _CB), jnp.int32),
        pltpu.VMEM((2, _CB, _CW), jnp.float32),
        pltpu.VMEM((_SB, _CW), jnp.float32),
        pltpu.VMEM((_SB, _CW), jnp.float32),
        pltpu.VMEM_SHARED((_AROWS, _CW), jnp.float32),
        pltpu.SemaphoreType.DMA,
        pltpu.SemaphoreType.DMA,
    ]

    @functools.partial(pl.kernel, out_type=out_type, mesh=mesh,
                       scratch_types=scratch)
    def k(*refs):
        tabs = refs[:npass]
        sh, dlo, dhi, zh, onesh = refs[npass:npass + 5]
        s_outs = refs[npass + 5:npass + 5 + ntot]
        (srcv, dstv, rows, zbuf, wbuf, acc,
         sem0, sem1) = refs[npass + 5 + ntot:]
        sems = (sem0, sem1)

        cid = lax.axis_index("c")
        sid = lax.axis_index("s")
        base = cid * _HN
        pltpu.sync_copy(zh, zbuf)

        for p in range(npass):
            _zero_acc(sid, acc, zbuf)
            plsc.subcore_barrier()

            @pl.when(cid == 0)
            def _(tab=tabs[p]):
                _edge_pass(sid, sh, dlo, tab, acc, srcv, dstv, rows, sems)

            @pl.when(cid == 1)
            def _(tab=tabs[p]):
                _edge_pass(sid, sh, dhi, tab, acc, srcv, dstv, rows, sems)
            plsc.subcore_barrier()
            _writeback(sid, acc, wbuf, s_outs[p], base)
            plsc.subcore_barrier()

        if with_counts:
            _zero_acc(sid, acc, zbuf)
            pltpu.sync_copy(onesh, rows.at[0])
            plsc.subcore_barrier()

            def cbody(kk, carry, dh):
                off = (sid + kk * _NSUB) * _CB
                pltpu.sync_copy(dh.at[pl.ds(off, _CB)], dstv.at[0])
                pltpu.sync_copy(rows.at[0], acc.at[dstv.at[0]], add=True)
                return carry

            @pl.when(cid == 0)
            def _():
                lax.fori_loop(0, _KPT, functools.partial(cbody, dh=dlo), 0)

            @pl.when(cid == 1)
            def _():
                lax.fori_loop(0, _KPT, functools.partial(cbody, dh=dhi), 0)
            plsc.subcore_barrier()
            _writeback(sid, acc, wbuf, s_outs[npass], base)

    args = list(tables) + [srcP, dloP, dhiP, zeros_in, ones128]
    res = k(*args)
    return tuple(res) if isinstance(res, (list, tuple)) else (res,)


_PROBE_STAGE = 3


def _sc_probe(table, srcP, dloP, zeros_in):
    mesh = plsc.VectorSubcoreMesh(core_axis_name="c", subcore_axis_name="s")
    scratch = [
        pltpu.VMEM((_CB,), jnp.int32),
        pltpu.VMEM((_CB,), jnp.int32),
        pltpu.VMEM((_CB, _CW), jnp.float32),
        pltpu.VMEM((_SB, _CW), jnp.float32),
        pltpu.VMEM((_SB, _CW), jnp.float32),
        pltpu.VMEM_SHARED((_AROWS, _CW), jnp.float32),
        pltpu.SemaphoreType.DMA,
    ]

    @functools.partial(
        pl.kernel,
        out_type=jax.ShapeDtypeStruct((N, _CW), jnp.float32),
        mesh=mesh, scratch_types=scratch)
    def k(tab, sh, dlo, zh, s_out, srcv, dstv, rows, zbuf, wbuf, acc, sem):
        cid = lax.axis_index("c")
        sid = lax.axis_index("s")
        base = cid * _HN
        pltpu.sync_copy(zh, zbuf)
        _zero_acc(sid, acc, zbuf)
        plsc.subcore_barrier()

        if _PROBE_STAGE >= 2:
            def body(kk, carry):
                off = (sid + kk * _NSUB) * _CB
                pltpu.sync_copy(sh.at[pl.ds(off, _CB)], srcv)
                pltpu.sync_copy(dlo.at[pl.ds(off, _CB)], dstv)
                pltpu.async_copy(tab.at[srcv], rows, sem).wait()
                if _PROBE_STAGE >= 3:
                    pltpu.sync_copy(rows, acc.at[dstv], add=True)
                return carry

            lax.fori_loop(0, _KPT, body, 0)
        plsc.subcore_barrier()
        _writeback(sid, acc, wbuf, s_out, base)

    return k(table, srcP, dloP, zeros_in)


def _tc_localize(dstP):
    def body(d_ref, lo_ref, hi_ref):
        d = d_ref[...]
        lo_ref[...] = jnp.where(d < _HN, d, _TRASH)
        hi_ref[...] = jnp.where((d >= _HN) & (d < N), d - _HN, _TRASH)

    return pl.pallas_call(
        body,
        out_shape=[jax.ShapeDtypeStruct((_EPAD // _CB, _CB), jnp.int32)] * 2,
    )(dstP)


def _tc_layer0(s0, cntb, x, Wl0T, Wr0T, bl0, g0, b0):
    def body(s0_ref, cnt_ref, x_ref, wl_ref, wr_ref, bl_ref, g_ref, b_ref,
             ofull, *ochunks):
        inv = 1.0 / jnp.maximum(cnt_ref[...], 1.0)
        zl = jnp.dot(s0_ref[...], wl_ref[...],
                     preferred_element_type=jnp.float32)
        zr = jnp.dot(x_ref[...], wr_ref[...],
                     preferred_element_type=jnp.float32)
        z = jnp.concatenate(
            [zl[:, c * 128:(c + 1) * 128] * inv for c in range(H // 128)],
            axis=1)
        z = ((z + zr + bl_ref[...]) * (g_ref[...] * _BNS)) + b_ref[...]
        h = jnp.maximum(z, 0.0)
        ofull[...] = h
        for j, o in enumerate(ochunks):
            o[...] = h[:, j * _CW:(j + 1) * _CW]

    return pl.pallas_call(
        body,
        grid=(_NB,),
        in_specs=[
            pl.BlockSpec((_R, IN), lambda i: (i, 0)),
            pl.BlockSpec((_R, 128), lambda i: (i, 0)),
            pl.BlockSpec((_R, IN), lambda i: (i, 0)),
            pl.BlockSpec((IN, H), lambda i: (0, 0)),
            pl.BlockSpec((IN, H), lambda i: (0, 0)),
            pl.BlockSpec((1, H), lambda i: (0, 0)),
            pl.BlockSpec((1, H), lambda i: (0, 0)),
            pl.BlockSpec((1, H), lambda i: (0, 0)),
        ],
        out_specs=[pl.BlockSpec((_R, H), lambda i: (i, 0))] +
                  [pl.BlockSpec((_R, _CW), lambda i: (i, 0))] * (H // _CW),
        out_shape=[jax.ShapeDtypeStruct((N, H), jnp.float32)] +
                  [jax.ShapeDtypeStruct((N, _CW), jnp.float32)] * (H // _CW),
    )(s0, cntb, x, Wl0T, Wr0T, bl0, g0, b0)


def _tc_layer1(s1, hfull, cntb, batch3, Wl1T, Wr1T, bl1, g1, b1, WlinT, blin):
    def body(s1_ref, h_ref, cnt_ref, bt_ref, wl_ref, wr_ref,
             bl_ref, g_ref, b_ref, wo_ref, bo_ref, out_ref, pacc, pcacc):
        i = pl.program_id(0)
        inv = 1.0 / jnp.maximum(cnt_ref[...], 1.0)
        zl = jnp.dot(s1_ref[...], wl_ref[...],
                     preferred_element_type=jnp.float32)
        zr = jnp.dot(h_ref[...], wr_ref[...],
                     preferred_element_type=jnp.float32)
        z = jnp.concatenate(
            [zl[:, c * 128:(c + 1) * 128] * inv for c in range(H // 128)],
            axis=1)
        z = ((z + zr + bl_ref[...]) * (g_ref[...] * _BNS)) + b_ref[...]
        h = jnp.maximum(z, 0.0)

        bt = bt_ref[0]
        rows_iota = lax.broadcasted_iota(jnp.int32, (G, _R), 0)
        oh = (rows_iota == bt).astype(jnp.float32)
        ps = jnp.dot(oh, h, preferred_element_type=jnp.float32)
        pc = jnp.sum(oh, axis=1, keepdims=True)

        @pl.when(i == 0)
        def _():
            pacc[...] = jnp.zeros_like(pacc)
            pcacc[...] = jnp.zeros_like(pcacc)

        pacc[...] += ps
        pcacc[...] += jnp.broadcast_to(pc, (G, 128))

        @pl.when(i == _NB - 1)
        def _():
            invp = 1.0 / jnp.maximum(pcacc[...], 1.0)
            o = bo_ref[...]
            for c in range(H // 128):
                o += jnp.dot(pacc[:, c * 128:(c + 1) * 128] * invp,
                             wo_ref[c * 128:(c + 1) * 128, :],
                             preferred_element_type=jnp.float32)
            out_ref[...] = o

    return pl.pallas_call(
        body,
        grid=(_NB,),
        in_specs=[
            pl.BlockSpec((_R, H), lambda i: (i, 0)),
            pl.BlockSpec((_R, H), lambda i: (i, 0)),
            pl.BlockSpec((_R, 128), lambda i: (i, 0)),
            pl.BlockSpec((1, 1, _R), lambda i: (i, 0, 0)),
            pl.BlockSpec((H, H), lambda i: (0, 0)),
            pl.BlockSpec((H, H), lambda i: (0, 0)),
            pl.BlockSpec((1, H), lambda i: (0, 0)),
            pl.BlockSpec((1, H), lambda i: (0, 0)),
            pl.BlockSpec((1, H), lambda i: (0, 0)),
            pl.BlockSpec((H, OUT), lambda i: (0, 0)),
            pl.BlockSpec((1, OUT), lambda i: (0, 0)),
        ],
        out_specs=pl.BlockSpec((G, OUT), lambda i: (0, 0)),
        out_shape=jax.ShapeDtypeStruct((G, OUT), jnp.float32),
        scratch_shapes=[pltpu.VMEM((G, H), jnp.float32),
                        pltpu.VMEM((G, 128), jnp.float32)],
    )(s1, hfull, cntb, batch3, Wl1T, Wr1T, bl1, g1, b1, WlinT, blin)


def kernel(x, edge_index, batch, Wl0, bl0, Wr0, Wl1, bl1, Wr1,
           g0, b0, g1, b1, Wlin, blin):
    src = edge_index[0]
    dst = edge_index[1]
    srcP = jnp.concatenate([src, jnp.zeros((_EPAD - E,), jnp.int32)])
    dstP = jnp.concatenate([dst, jnp.full((_EPAD - E,), N, jnp.int32)])
    dlo, dhi = _tc_localize(dstP.reshape(_EPAD // _CB, _CB))
    dlo, dhi = dlo.reshape(_EPAD), dhi.reshape(_EPAD)
    xc = [x[:, j * _CW:(j + 1) * _CW] for j in range(IN // _CW)]
    zeros_in = jnp.zeros((_SB, _CW), jnp.float32)
    ones128 = jnp.ones((_CB, _CW), jnp.float32)

    res0 = _sc_segsum(xc, srcP, dlo, dhi, zeros_in, ones128, True)
    s0 = jnp.concatenate(res0[:IN // _CW], axis=1)
    cntb = res0[IN // _CW]
    h = _tc_layer0(s0, cntb, x, Wl0.T, Wr0.T,
                   bl0[None, :], g0[None, :], b0[None, :])
    hfull, hc = h[0], h[1:]
    res1 = _sc_segsum(list(hc), srcP, dlo, dhi, zeros_in, ones128, False)
    s1 = jnp.concatenate(res1, axis=1)
    out = _tc_layer1(s1, hfull, cntb, batch.reshape(_NB, 1, _R),
                     Wl1.T, Wr1.T, bl1[None, :], g1[None, :], b1[None, :],
                     Wlin.T, blin[None, :])
    return out


# speedup vs baseline: 1.2807x; 1.0871x over previous
import functools

import jax
import jax.numpy as jnp
from jax import lax
from jax.experimental import pallas as pl
from jax.experimental.pallas import tpu as pltpu
from jax.experimental.pallas import tpu_sc as plsc

N = 10000
E = 160000
IN = 256
H = 512
OUT = 128
G = 64
EPS = 1e-5
_BNS = (1.0 + EPS) ** -0.5

_CW = 128
_CB = 128
_NSUB = 16
_EPAD = 1280 * _CB
_KPT = _EPAD // _CB // _NSUB
_HN = N // 2
_TRASH = _HN
_AROWS = 5120
_RB = _AROWS // _NSUB
_SB = 64
_WLAST = _HN - (_NSUB - 1) * _RB
_R = 1000
_NB = N // _R


def _zero_acc(sid, acc, zbuf):
    for q in range(_RB // _SB):
        pltpu.sync_copy(zbuf, acc.at[pl.ds(sid * _RB + q * _SB, _SB)])


def _writeback(sid, acc, wbuf, out, base):
    def blk(r0, nr, woff):
        pltpu.sync_copy(acc.at[pl.ds(r0, nr)], wbuf.at[pl.ds(woff, nr)])
        pltpu.sync_copy(wbuf.at[pl.ds(woff, nr)], out.at[pl.ds(base + r0, nr)])

    @pl.when(sid < _NSUB - 1)
    def _():
        for q in range(_RB // _SB):
            blk(sid * _RB + q * _SB, _SB, 0)

    @pl.when(sid == _NSUB - 1)
    def _():
        r0 = (_NSUB - 1) * _RB
        for q in range(_WLAST // _SB):
            blk(r0 + q * _SB, _SB, 0)
        tail = _WLAST - (_WLAST // _SB) * _SB
        if tail:
            blk(r0 + (_WLAST // _SB) * _SB, tail, 0)


def _edge_pass(table, acc, srcv, dstv, rows, gsems, ssems):
    def fire_gather(k, b):
        pltpu.async_copy(table.at[srcv.at[k]], rows.at[b], gsems[b])

    def wait_gather(k, b):
        pltpu.make_async_copy(table.at[srcv.at[k]], rows.at[b],
                              gsems[b]).wait()

    def fire_scatter(k, b):
        pltpu.async_copy(rows.at[b], acc.at[dstv.at[k]], ssems[b], add=True)

    def wait_scatter(k, b):
        pltpu.make_async_copy(rows.at[b], acc.at[dstv.at[k]],
                              ssems[b]).wait()

    fire_gather(0, 0)
    fire_gather(1, 1)
    fire_gather(2, 2)

    def body(q, carry):
        k0 = q * 3
        for b in range(3):
            k = k0 + b
            wait_gather(k, b)
            fire_scatter(k, b)

            @pl.when(k >= 1)
            def _(k=k, b2=(b + 2) % 3):
                wait_scatter(k - 1, b2)
                fire_gather(k + 2, b2)
        return carry

    lax.fori_loop(0, (_KPT - 2) // 3, body, 0)
    for k, b in ((_KPT - 2, (_KPT - 2) % 3), (_KPT - 1, (_KPT - 1) % 3)):
        wait_gather(k, b)
        fire_scatter(k, b)
    for k in range(_KPT - 3, _KPT):
        wait_scatter(k, k % 3)


def _sc_segsum(tables, srcP, dloP, dhiP, zeros_in, ones128, with_counts):
    npass = len(tables)
    ntot = npass + (1 if with_counts else 0)
    mesh = plsc.VectorSubcoreMesh(core_axis_name="c", subcore_axis_name="s")
    out_type = [jax.ShapeDtypeStruct((N, _CW), jnp.float32)] * ntot
    scratch = [
        pltpu.VMEM((_KPT, _CB), jnp.int32),
        pltpu.VMEM((_KPT, _CB), jnp.int32),
        pltpu.VMEM((3, _CB, _CW), jnp.float32),
        pltpu.VMEM((_SB, _CW), jnp.float32),
        pltpu.VMEM((_SB, _CW), jnp.float32),
        pltpu.VMEM_SHARED((_AROWS, _CW), jnp.float32),
        pltpu.SemaphoreType.DMA,
        pltpu.SemaphoreType.DMA,
        pltpu.SemaphoreType.DMA,
        pltpu.SemaphoreType.DMA,
        pltpu.SemaphoreType.DMA,
        pltpu.SemaphoreType.DMA,
    ]

    @functools.partial(pl.kernel, out_type=out_type, mesh=mesh,
                       scratch_types=scratch)
    def k(*refs):
        tabs = refs[:npass]
        sh, dlo, dhi, zh, onesh = refs[npass:npass + 5]
        s_outs = refs[npass + 5:npass + 5 + ntot]
        (srcv, dstv, rows, zbuf, wbuf, acc,
         g0s, g1s, g2s, s0s, s1s, s2s) = refs[npass + 5 + ntot:]
        gsems = (g0s, g1s, g2s)
        ssems = (s0s, s1s, s2s)

        cid = lax.axis_index("c")
        sid = lax.axis_index("s")
        base = cid * _HN
        pltpu.sync_copy(zh, zbuf)
        pltpu.sync_copy(sh.at[pl.ds(sid * _KPT, _KPT)], srcv)

        @pl.when(cid == 0)
        def _():
            pltpu.sync_copy(dlo.at[pl.ds(sid * _KPT, _KPT)], dstv)

        @pl.when(cid == 1)
        def _():
            pltpu.sync_copy(dhi.at[pl.ds(sid * _KPT, _KPT)], dstv)

        for p in range(npass):
            _zero_acc(sid, acc, zbuf)
            plsc.subcore_barrier()
            _edge_pass(tabs[p], acc, srcv, dstv, rows, gsems, ssems)
            plsc.subcore_barrier()
            _writeback(sid, acc, wbuf, s_outs[p], base)
            plsc.subcore_barrier()

        if with_counts:
            _zero_acc(sid, acc, zbuf)
            pltpu.sync_copy(onesh, rows.at[0])
            plsc.subcore_barrier()
            ones_src = rows.at[0]

            def cfire(kk, b):
                pltpu.async_copy(ones_src, acc.at[dstv.at[kk]], ssems[b],
                                 add=True)

            def cwait(kk, b):
                pltpu.make_async_copy(ones_src, acc.at[dstv.at[kk]],
                                      ssems[b]).wait()

            cfire(0, 0)
            cfire(1, 1)

            def cbody(q, carry):
                for b in range(2):
                    kk = q * 2 + b
                    cwait(kk, b)

                    @pl.when(kk + 2 < _KPT)
                    def _(kk=kk, b=b):
                        cfire(kk + 2, b)
                return carry

            lax.fori_loop(0, _KPT // 2, cbody, 0)
            plsc.subcore_barrier()
            _writeback(sid, acc, wbuf, s_outs[npass], base)

    args = list(tables) + [srcP, dloP, dhiP, zeros_in, ones128]
    res = k(*args)
    return tuple(res) if isinstance(res, (list, tuple)) else (res,)


_PROBE_STAGE = 3


def _sc_probe(table, srcP, dloP, zeros_in):
    mesh = plsc.VectorSubcoreMesh(core_axis_name="c", subcore_axis_name="s")
    scratch = [
        pltpu.VMEM((_CB,), jnp.int32),
        pltpu.VMEM((_CB,), jnp.int32),
        pltpu.VMEM((_CB, _CW), jnp.float32),
        pltpu.VMEM((_SB, _CW), jnp.float32),
        pltpu.VMEM((_SB, _CW), jnp.float32),
        pltpu.VMEM_SHARED((_AROWS, _CW), jnp.float32),
        pltpu.SemaphoreType.DMA,
    ]

    @functools.partial(
        pl.kernel,
        out_type=jax.ShapeDtypeStruct((N, _CW), jnp.float32),
        mesh=mesh, scratch_types=scratch)
    def k(tab, sh, dlo, zh, s_out, srcv, dstv, rows, zbuf, wbuf, acc, sem):
        cid = lax.axis_index("c")
        sid = lax.axis_index("s")
        base = cid * _HN
        pltpu.sync_copy(zh, zbuf)
        _zero_acc(sid, acc, zbuf)
        plsc.subcore_barrier()

        if _PROBE_STAGE >= 2:
            def body(kk, carry):
                off = (sid + kk * _NSUB) * _CB
                pltpu.sync_copy(sh.at[pl.ds(off, _CB)], srcv)
                pltpu.sync_copy(dlo.at[pl.ds(off, _CB)], dstv)
                pltpu.async_copy(tab.at[srcv], rows, sem).wait()
                if _PROBE_STAGE >= 3:
                    pltpu.sync_copy(rows, acc.at[dstv], add=True)
                return carry

            lax.fori_loop(0, _KPT, body, 0)
        plsc.subcore_barrier()
        _writeback(sid, acc, wbuf, s_out, base)

    return k(table, srcP, dloP, zeros_in)


def _tc_localize(dstP):
    def body(d_ref, lo_ref, hi_ref):
        d = d_ref[...]
        lo_ref[...] = jnp.where(d < _HN, d, _TRASH)
        hi_ref[...] = jnp.where((d >= _HN) & (d < N), d - _HN, _TRASH)

    return pl.pallas_call(
        body,
        out_shape=[jax.ShapeDtypeStruct((_EPAD // _CB, _CB), jnp.int32)] * 2,
    )(dstP)


def _tc_layer0(s0, cntb, x, Wl0T, Wr0T, bl0, g0, b0):
    def body(s0_ref, cnt_ref, x_ref, wl_ref, wr_ref, bl_ref, g_ref, b_ref,
             ofull, *ochunks):
        inv = 1.0 / jnp.maximum(cnt_ref[...], 1.0)
        zl = jnp.dot(s0_ref[...], wl_ref[...],
                     preferred_element_type=jnp.float32)
        zr = jnp.dot(x_ref[...], wr_ref[...],
                     preferred_element_type=jnp.float32)
        z = jnp.concatenate(
            [zl[:, c * 128:(c + 1) * 128] * inv for c in range(H // 128)],
            axis=1)
        z = ((z + zr + bl_ref[...]) * (g_ref[...] * _BNS)) + b_ref[...]
        h = jnp.maximum(z, 0.0)
        ofull[...] = h
        for j, o in enumerate(ochunks):
            o[...] = h[:, j * _CW:(j + 1) * _CW]

    return pl.pallas_call(
        body,
        grid=(_NB,),
        in_specs=[
            pl.BlockSpec((_R, IN), lambda i: (i, 0)),
            pl.BlockSpec((_R, 128), lambda i: (i, 0)),
            pl.BlockSpec((_R, IN), lambda i: (i, 0)),
            pl.BlockSpec((IN, H), lambda i: (0, 0)),
            pl.BlockSpec((IN, H), lambda i: (0, 0)),
            pl.BlockSpec((1, H), lambda i: (0, 0)),
            pl.BlockSpec((1, H), lambda i: (0, 0)),
            pl.BlockSpec((1, H), lambda i: (0, 0)),
        ],
        out_specs=[pl.BlockSpec((_R, H), lambda i: (i, 0))] +
                  [pl.BlockSpec((_R, _CW), lambda i: (i, 0))] * (H // _CW),
        out_shape=[jax.ShapeDtypeStruct((N, H), jnp.float32)] +
                  [jax.ShapeDtypeStruct((N, _CW), jnp.float32)] * (H // _CW),
    )(s0, cntb, x, Wl0T, Wr0T, bl0, g0, b0)


def _tc_layer1(s1, hfull, cntb, batch3, Wl1T, Wr1T, bl1, g1, b1, WlinT, blin):
    def body(s1_ref, h_ref, cnt_ref, bt_ref, wl_ref, wr_ref,
             bl_ref, g_ref, b_ref, wo_ref, bo_ref, out_ref, pacc, pcacc):
        i = pl.program_id(0)
        inv = 1.0 / jnp.maximum(cnt_ref[...], 1.0)
        zl = jnp.dot(s1_ref[...], wl_ref[...],
                     preferred_element_type=jnp.float32)
        zr = jnp.dot(h_ref[...], wr_ref[...],
                     preferred_element_type=jnp.float32)
        z = jnp.concatenate(
            [zl[:, c * 128:(c + 1) * 128] * inv for c in range(H // 128)],
            axis=1)
        z = ((z + zr + bl_ref[...]) * (g_ref[...] * _BNS)) + b_ref[...]
        h = jnp.maximum(z, 0.0)

        bt = bt_ref[0]
        rows_iota = lax.broadcasted_iota(jnp.int32, (G, _R), 0)
        oh = (rows_iota == bt).astype(jnp.float32)
        ps = jnp.dot(oh, h, preferred_element_type=jnp.float32)
        pc = jnp.sum(oh, axis=1, keepdims=True)

        @pl.when(i == 0)
        def _():
            pacc[...] = jnp.zeros_like(pacc)
            pcacc[...] = jnp.zeros_like(pcacc)

        pacc[...] += ps
        pcacc[...] += jnp.broadcast_to(pc, (G, 128))

        @pl.when(i == _NB - 1)
        def _():
            invp = 1.0 / jnp.maximum(pcacc[...], 1.0)
            o = bo_ref[...]
            for c in range(H // 128):
                o += jnp.dot(pacc[:, c * 128:(c + 1) * 128] * invp,
                             wo_ref[c * 128:(c + 1) * 128, :],
                             preferred_element_type=jnp.float32)
            out_ref[...] = o

    return pl.pallas_call(
        body,
        grid=(_NB,),
        in_specs=[
            pl.BlockSpec((_R, H), lambda i: (i, 0)),
            pl.BlockSpec((_R, H), lambda i: (i, 0)),
            pl.BlockSpec((_R, 128), lambda i: (i, 0)),
            pl.BlockSpec((1, 1, _R), lambda i: (i, 0, 0)),
            pl.BlockSpec((H, H), lambda i: (0, 0)),
            pl.BlockSpec((H, H), lambda i: (0, 0)),
            pl.BlockSpec((1, H), lambda i: (0, 0)),
            pl.BlockSpec((1, H), lambda i: (0, 0)),
            pl.BlockSpec((1, H), lambda i: (0, 0)),
            pl.BlockSpec((H, OUT), lambda i: (0, 0)),
            pl.BlockSpec((1, OUT), lambda i: (0, 0)),
        ],
        out_specs=pl.BlockSpec((G, OUT), lambda i: (0, 0)),
        out_shape=jax.ShapeDtypeStruct((G, OUT), jnp.float32),
        scratch_shapes=[pltpu.VMEM((G, H), jnp.float32),
                        pltpu.VMEM((G, 128), jnp.float32)],
    )(s1, hfull, cntb, batch3, Wl1T, Wr1T, bl1, g1, b1, WlinT, blin)


def kernel(x, edge_index, batch, Wl0, bl0, Wr0, Wl1, bl1, Wr1,
           g0, b0, g1, b1, Wlin, blin):
    src = edge_index[0]
    dst = edge_index[1]
    srcP = jnp.concatenate([src, jnp.zeros((_EPAD - E,), jnp.int32)])
    srcP = srcP.reshape(_EPAD // _CB, _CB)
    dstP = jnp.concatenate([dst, jnp.full((_EPAD - E,), N, jnp.int32)])
    dlo, dhi = _tc_localize(dstP.reshape(_EPAD // _CB, _CB))
    xc = [x[:, j * _CW:(j + 1) * _CW] for j in range(IN // _CW)]
    zeros_in = jnp.zeros((_SB, _CW), jnp.float32)
    ones128 = jnp.ones((_CB, _CW), jnp.float32)

    res0 = _sc_segsum(xc, srcP, dlo, dhi, zeros_in, ones128, True)
    s0 = jnp.concatenate(res0[:IN // _CW], axis=1)
    cntb = res0[IN // _CW]
    h = _tc_layer0(s0, cntb, x, Wl0.T, Wr0.T,
                   bl0[None, :], g0[None, :], b0[None, :])
    hfull, hc = h[0], h[1:]
    res1 = _sc_segsum(list(hc), srcP, dlo, dhi, zeros_in, ones128, False)
    s1 = jnp.concatenate(res1, axis=1)
    out = _tc_layer1(s1, hfull, cntb, batch.reshape(_NB, 1, _R),
                     Wl1.T, Wr1.T, bl1[None, :], g1[None, :], b1[None, :],
                     Wlin.T, blin[None, :])
    return out


# speedup vs baseline: 2.2618x; 1.7660x over previous
import functools

import jax
import jax.numpy as jnp
from jax import lax
from jax.experimental import pallas as pl
from jax.experimental.pallas import tpu as pltpu
from jax.experimental.pallas import tpu_sc as plsc

N = 10000
E = 160000
IN = 256
H = 512
OUT = 128
G = 64
EPS = 1e-5
_BNS = (1.0 + EPS) ** -0.5

_CW = 128
_CB = 128
_NSUB = 16
_EPAD = 1280 * _CB
_KPT = _EPAD // _CB // _NSUB // 2
_AROWS = 10240
_RB = _AROWS // _NSUB
_SB = 16
_WLAST = N - (_NSUB - 1) * _RB
_R = 1000
_NB = N // _R


def _zero_acc(sid, acc, zbuf):
    def body(q, carry):
        pltpu.sync_copy(zbuf, acc.at[pl.ds(sid * _RB + q * _SB, _SB)])
        return carry

    lax.fori_loop(0, _RB // _SB, body, 0)


def _writeback(sid, acc, wbuf, out):
    def blk(r0, carry, nr=_SB):
        pltpu.sync_copy(acc.at[pl.ds(r0, nr)], wbuf.at[pl.ds(0, nr)])
        pltpu.sync_copy(wbuf.at[pl.ds(0, nr)], out.at[pl.ds(r0, nr)])
        return carry

    @pl.when(sid < _NSUB - 1)
    def _():
        lax.fori_loop(0, _RB // _SB,
                      lambda q, c: blk(sid * _RB + q * _SB, c), 0)

    @pl.when(sid == _NSUB - 1)
    def _():
        r0 = (_NSUB - 1) * _RB
        lax.fori_loop(0, _WLAST // _SB,
                      lambda q, c: blk(r0 + q * _SB, c), 0)


def _edge_pass(table, acc, srcv, dstv, rows, gsems, ssems):
    def fire_gather(k, b):
        pltpu.async_copy(table.at[srcv.at[k]], rows.at[b], gsems[b])

    def wait_gather(k, b):
        pltpu.make_async_copy(table.at[srcv.at[k]], rows.at[b],
                              gsems[b]).wait()

    def fire_scatter(k, b):
        pltpu.async_copy(rows.at[b], acc.at[dstv.at[k]], ssems[b], add=True)

    def wait_scatter(k, b):
        pltpu.make_async_copy(rows.at[b], acc.at[dstv.at[k]],
                              ssems[b]).wait()

    fire_gather(0, 0)
    fire_gather(1, 1)

    def body(q, carry):
        for b in range(2):
            k = q * 2 + b
            wait_gather(k, b)
            fire_scatter(k, b)
            wait_scatter(k, b)

            @pl.when(k + 2 < _KPT)
            def _(k=k, b=b):
                fire_gather(k + 2, b)
        return carry

    lax.fori_loop(0, _KPT // 2, body, 0)


def _sc_segsum(tables, srcP, dstP, zeros_in, ones128, with_counts):
    npass = len(tables)
    ntot = 2 * npass + (2 if with_counts else 0)
    mesh = plsc.VectorSubcoreMesh(core_axis_name="c", subcore_axis_name="s")
    out_type = [jax.ShapeDtypeStruct((N, _CW), jnp.float32)] * ntot
    scratch = [
        pltpu.VMEM((_KPT, _CB), jnp.int32),
        pltpu.VMEM((_KPT, _CB), jnp.int32),
        pltpu.VMEM((2, _CB, _CW), jnp.float32),
        pltpu.VMEM((_SB, _CW), jnp.float32),
        pltpu.VMEM((_SB, _CW), jnp.float32),
        pltpu.VMEM_SHARED((_AROWS, _CW), jnp.float32),
        pltpu.SemaphoreType.DMA,
        pltpu.SemaphoreType.DMA,
        pltpu.SemaphoreType.DMA,
        pltpu.SemaphoreType.DMA,
    ]

    @functools.partial(pl.kernel, out_type=out_type, mesh=mesh,
                       scratch_types=scratch)
    def k(*refs):
        tabs = refs[:npass]
        sh, dh, zh, onesh = refs[npass:npass + 4]
        s_outs = refs[npass + 4:npass + 4 + ntot]
        (srcv, dstv, rows, zbuf, wbuf, acc,
         g0s, g1s, s0s, s1s) = refs[npass + 4 + ntot:]
        gsems = (g0s, g1s)
        ssems = (s0s, s1s)

        cid = lax.axis_index("c")
        sid = lax.axis_index("s")
        pltpu.sync_copy(zh, zbuf)
        coff = (cid * _NSUB + sid) * _KPT
        pltpu.sync_copy(sh.at[pl.ds(coff, _KPT)], srcv)
        pltpu.sync_copy(dh.at[pl.ds(coff, _KPT)], dstv)

        def wb(outs):
            @pl.when(cid == 0)
            def _():
                _writeback(sid, acc, wbuf, outs[0])

            @pl.when(cid == 1)
            def _():
                _writeback(sid, acc, wbuf, outs[1])

        for p in range(npass):
            _zero_acc(sid, acc, zbuf)
            plsc.subcore_barrier()
            _edge_pass(tabs[p], acc, srcv, dstv, rows, gsems, ssems)
            plsc.subcore_barrier()
            wb(s_outs[2 * p:2 * p + 2])
            plsc.subcore_barrier()

        if with_counts:
            _zero_acc(sid, acc, zbuf)
            pltpu.sync_copy(onesh, rows.at[0])
            plsc.subcore_barrier()
            ones_src = rows.at[0]

            def cfire(kk, b):
                pltpu.async_copy(ones_src, acc.at[dstv.at[kk]], ssems[b],
                                 add=True)

            def cwait(kk, b):
                pltpu.make_async_copy(ones_src, acc.at[dstv.at[kk]],
                                      ssems[b]).wait()

            cfire(0, 0)
            cfire(1, 1)

            def cbody(q, carry):
                for b in range(2):
                    kk = q * 2 + b
                    cwait(kk, b)

                    @pl.when(kk + 2 < _KPT)
                    def _(kk=kk, b=b):
                        cfire(kk + 2, b)
                return carry

            lax.fori_loop(0, _KPT // 2, cbody, 0)
            plsc.subcore_barrier()
            wb(s_outs[2 * npass:2 * npass + 2])

    args = list(tables) + [srcP, dstP, zeros_in, ones128]
    res = k(*args)
    return tuple(res) if isinstance(res, (list, tuple)) else (res,)


_PROBE_STAGE = 3


def _sc_probe(table, srcP, dloP, zeros_in):
    mesh = plsc.VectorSubcoreMesh(core_axis_name="c", subcore_axis_name="s")
    scratch = [
        pltpu.VMEM((_CB,), jnp.int32),
        pltpu.VMEM((_CB,), jnp.int32),
        pltpu.VMEM((_CB, _CW), jnp.float32),
        pltpu.VMEM((_SB, _CW), jnp.float32),
        pltpu.VMEM((_SB, _CW), jnp.float32),
        pltpu.VMEM_SHARED((_AROWS, _CW), jnp.float32),
        pltpu.SemaphoreType.DMA,
    ]

    @functools.partial(
        pl.kernel,
        out_type=jax.ShapeDtypeStruct((N, _CW), jnp.float32),
        mesh=mesh, scratch_types=scratch)
    def k(tab, sh, dlo, zh, s_out, srcv, dstv, rows, zbuf, wbuf, acc, sem):
        cid = lax.axis_index("c")
        sid = lax.axis_index("s")
        base = cid * _HN
        pltpu.sync_copy(zh, zbuf)
        _zero_acc(sid, acc, zbuf)
        plsc.subcore_barrier()

        if _PROBE_STAGE >= 2:
            def body(kk, carry):
                off = (sid + kk * _NSUB) * _CB
                pltpu.sync_copy(sh.at[pl.ds(off, _CB)], srcv)
                pltpu.sync_copy(dlo.at[pl.ds(off, _CB)], dstv)
                pltpu.async_copy(tab.at[srcv], rows, sem).wait()
                if _PROBE_STAGE >= 3:
                    pltpu.sync_copy(rows, acc.at[dstv], add=True)
                return carry

            lax.fori_loop(0, _KPT, body, 0)
        plsc.subcore_barrier()
        _writeback(sid, acc, wbuf, s_out, base)

    return k(table, srcP, dloP, zeros_in)


def _tc_layer0(s0p, cntp, x, Wl0T, Wr0T, bl0, g0, b0):
    def body(a0, b0_, a1, b1_, ca, cb, x_ref, wl_ref, wr_ref, bl_ref,
             g_ref, b_ref, ofull, *ochunks):
        inv = 1.0 / jnp.maximum(ca[...] + cb[...], 1.0)
        s0 = jnp.concatenate([a0[...] + b0_[...], a1[...] + b1_[...]],
                             axis=1)
        zl = jnp.dot(s0, wl_ref[...], preferred_element_type=jnp.float32)
        zr = jnp.dot(x_ref[...], wr_ref[...],
                     preferred_element_type=jnp.float32)
        z = jnp.concatenate(
            [zl[:, c * 128:(c + 1) * 128] * inv for c in range(H // 128)],
            axis=1)
        z = ((z + zr + bl_ref[...]) * (g_ref[...] * _BNS)) + b_ref[...]
        h = jnp.maximum(z, 0.0)
        ofull[...] = h
        for j, o in enumerate(ochunks):
            o[...] = h[:, j * _CW:(j + 1) * _CW]

    return pl.pallas_call(
        body,
        grid=(_NB,),
        in_specs=[pl.BlockSpec((_R, 128), lambda i: (i, 0))] * 6 + [
            pl.BlockSpec((_R, IN), lambda i: (i, 0)),
            pl.BlockSpec((IN, H), lambda i: (0, 0)),
            pl.BlockSpec((IN, H), lambda i: (0, 0)),
            pl.BlockSpec((1, H), lambda i: (0, 0)),
            pl.BlockSpec((1, H), lambda i: (0, 0)),
            pl.BlockSpec((1, H), lambda i: (0, 0)),
        ],
        out_specs=[pl.BlockSpec((_R, H), lambda i: (i, 0))] +
                  [pl.BlockSpec((_R, _CW), lambda i: (i, 0))] * (H // _CW),
        out_shape=[jax.ShapeDtypeStruct((N, H), jnp.float32)] +
                  [jax.ShapeDtypeStruct((N, _CW), jnp.float32)] * (H // _CW),
    )(s0p[0], s0p[1], s0p[2], s0p[3], cntp[0], cntp[1],
      x, Wl0T, Wr0T, bl0, g0, b0)


def _tc_layer1(s1p, hfull, cntp, batch3, Wl1T, Wr1T, bl1, g1, b1,
               WlinT, blin):
    def body(p0, q0, p1, q1, p2, q2, p3, q3, h_ref, ca, cb, bt_ref,
             wl_ref, wr_ref, bl_ref, g_ref, b_ref, wo_ref, bo_ref,
             out_ref, pacc, pcacc):
        i = pl.program_id(0)
        inv = 1.0 / jnp.maximum(ca[...] + cb[...], 1.0)
        s1 = jnp.concatenate([p0[...] + q0[...], p1[...] + q1[...],
                              p2[...] + q2[...], p3[...] + q3[...]], axis=1)
        zl = jnp.dot(s1, wl_ref[...], preferred_element_type=jnp.float32)
        zr = jnp.dot(h_ref[...], wr_ref[...],
                     preferred_element_type=jnp.float32)
        z = jnp.concatenate(
            [zl[:, c * 128:(c + 1) * 128] * inv for c in range(H // 128)],
            axis=1)
        z = ((z + zr + bl_ref[...]) * (g_ref[...] * _BNS)) + b_ref[...]
        h = jnp.maximum(z, 0.0)

        bt = bt_ref[0]
        rows_iota = lax.broadcasted_iota(jnp.int32, (G, _R), 0)
        oh = (rows_iota == bt).astype(jnp.float32)
        ps = jnp.dot(oh, h, preferred_element_type=jnp.float32)
        pc = jnp.sum(oh, axis=1, keepdims=True)

        @pl.when(i == 0)
        def _():
            pacc[...] = jnp.zeros_like(pacc)
            pcacc[...] = jnp.zeros_like(pcacc)

        pacc[...] += ps
        pcacc[...] += jnp.broadcast_to(pc, (G, 128))

        @pl.when(i == _NB - 1)
        def _():
            invp = 1.0 / jnp.maximum(pcacc[...], 1.0)
            o = bo_ref[...]
            for c in range(H // 128):
                o += jnp.dot(pacc[:, c * 128:(c + 1) * 128] * invp,
                             wo_ref[c * 128:(c + 1) * 128, :],
                             preferred_element_type=jnp.float32)
            out_ref[...] = o

    return pl.pallas_call(
        body,
        grid=(_NB,),
        in_specs=[pl.BlockSpec((_R, 128), lambda i: (i, 0))] * 8 + [
            pl.BlockSpec((_R, H), lambda i: (i, 0)),
            pl.BlockSpec((_R, 128), lambda i: (i, 0)),
            pl.BlockSpec((_R, 128), lambda i: (i, 0)),
            pl.BlockSpec((1, 1, _R), lambda i: (i, 0, 0)),
            pl.BlockSpec((H, H), lambda i: (0, 0)),
            pl.BlockSpec((H, H), lambda i: (0, 0)),
            pl.BlockSpec((1, H), lambda i: (0, 0)),
            pl.BlockSpec((1, H), lambda i: (0, 0)),
            pl.BlockSpec((1, H), lambda i: (0, 0)),
            pl.BlockSpec((H, OUT), lambda i: (0, 0)),
            pl.BlockSpec((1, OUT), lambda i: (0, 0)),
        ],
        out_specs=pl.BlockSpec((G, OUT), lambda i: (0, 0)),
        out_shape=jax.ShapeDtypeStruct((G, OUT), jnp.float32),
        scratch_shapes=[pltpu.VMEM((G, H), jnp.float32),
                        pltpu.VMEM((G, 128), jnp.float32)],
    )(s1p[0], s1p[1], s1p[2], s1p[3], s1p[4], s1p[5], s1p[6], s1p[7],
      hfull, cntp[0], cntp[1], batch3, Wl1T, Wr1T, bl1, g1, b1, WlinT, blin)


def kernel(x, edge_index, batch, Wl0, bl0, Wr0, Wl1, bl1, Wr1,
           g0, b0, g1, b1, Wlin, blin):
    src = edge_index[0]
    dst = edge_index[1]
    srcP = jnp.concatenate([src, jnp.zeros((_EPAD - E,), jnp.int32)])
    srcP = srcP.reshape(_EPAD // _CB, _CB)
    dstP = jnp.concatenate([dst, jnp.full((_EPAD - E,), N, jnp.int32)])
    dstP = dstP.reshape(_EPAD // _CB, _CB)
    xc = [x[:, j * _CW:(j + 1) * _CW] for j in range(IN // _CW)]
    zeros_in = jnp.zeros((_SB, _CW), jnp.float32)
    ones128 = jnp.ones((_CB, _CW), jnp.float32)

    res0 = _sc_segsum(xc, srcP, dstP, zeros_in, ones128, True)
    h = _tc_layer0(res0[:4], res0[4:6], x, Wl0.T, Wr0.T,
                   bl0[None, :], g0[None, :], b0[None, :])
    hfull, hc = h[0], h[1:]
    res1 = _sc_segsum(list(hc), srcP, dstP, zeros_in, ones128, False)
    out = _tc_layer1(res1, hfull, res0[4:6], batch.reshape(_NB, 1, _R),
                     Wl1.T, Wr1.T, bl1[None, :], g1[None, :], b1[None, :],
                     Wlin.T, blin[None, :])
    return out
